# baseline (speedup 1.0000x reference)
import functools

import jax
import jax.numpy as jnp
from jax import lax
from jax.experimental import pallas as pl
from jax.experimental.pallas import tpu as pltpu
from jax.experimental.pallas import tpu_sc as plsc

ATR_MULTIPLE = 2.0
L = 16


def _body(T, C, BPW,
          pos_hbm, prev_hbm, act_hbm, data_hbm, artr_hbm, date_hbm, time_hbm,
          out_hbm,
          pos_v, prev_v, act_v, date_v, time_v, idx_v, idx2_v, atrv_v,
          refp_v, out_v, sem1, sem2):
    nc = 2
    wid = lax.axis_index("s") * nc + lax.axis_index("c")
    base = wid * BPW

    pltpu.sync_copy(date_hbm.at[pl.ds(base, BPW)], date_v)
    pltpu.sync_copy(time_hbm.at[pl.ds(base, BPW)], time_v)
    pltpu.sync_copy(pos_hbm.at[pl.ds(base, BPW)], pos_v)
    pltpu.sync_copy(act_hbm.at[pl.ds(base, BPW)], act_v)
    pltpu.sync_copy(prev_hbm.at[pl.ds(base, BPW)], prev_v)

    def idx_body(i, carry):
        o = i * L
        d = date_v[pl.ds(o, L)]
        t = time_v[pl.ds(o, L)]
        p = pos_v[pl.ds(o, L)]
        a = act_v[pl.ds(o, L)]
        n = d * T + t
        idx_v[pl.ds(o, L)] = n
        direction = jnp.sign(p + a)
        chan = jnp.where(p == 0.0, 3,
                         jnp.where(direction > 0.0, 1, 2)).astype(jnp.int32)
        idx2_v[pl.ds(o, L)] = n * C + chan
        return carry

    lax.fori_loop(0, BPW // L, idx_body, 0)

    cp1 = pltpu.async_copy(artr_hbm.at[idx_v], atrv_v, sem1)
    cp2 = pltpu.async_copy(data_hbm.at[idx2_v], refp_v, sem2)
    cp1.wait()
    cp2.wait()

    def out_body(i, carry):
        o = i * L
        p = pos_v[pl.ds(o, L)]
        a = act_v[pl.ds(o, L)]
        prev0 = prev_v[pl.ds(o, L)]
        av = atrv_v[pl.ds(o, L)] * ATR_MULTIPLE + 1.0
        direction = jnp.sign(p + a)
        rp = refp_v[pl.ds(o, L)]
        prev = jnp.where(jnp.isnan(prev0) & (direction != 0.0),
                         -jnp.inf * direction, prev0)
        sp = jnp.where(direction > 0.0,
                       jnp.maximum(prev, rp / av),
                       jnp.minimum(prev, rp * av))
        sp = jnp.where(jnp.isnan(sp) | (direction == 0.0), prev, sp)
        out_v[pl.ds(o, L)] = sp
        return carry

    lax.fori_loop(0, BPW // L, out_body, 0)

    pltpu.sync_copy(out_v, out_hbm.at[pl.ds(base, BPW)])


@jax.jit
def kernel(position, prev_stop_loss, action, data, artr, date_idx, time_idx):
    D, T, C = data.shape
    B = position.shape[0]
    NW = 32
    BPW = B // NW

    data_flat = data.reshape(D * T * C)
    artr_flat = artr.reshape(D * T)
    date_i = date_idx.astype(jnp.int32)
    time_i = time_idx.astype(jnp.int32)

    mesh = plsc.VectorSubcoreMesh(core_axis_name="c", subcore_axis_name="s")
    run = pl.kernel(
        functools.partial(_body, T, C, BPW),
        out_type=jax.ShapeDtypeStruct((B,), jnp.float32),
        mesh=mesh,
        scratch_types=[
            pltpu.VMEM((BPW,), jnp.float32),
            pltpu.VMEM((BPW,), jnp.float32),
            pltpu.VMEM((BPW,), jnp.float32),
            pltpu.VMEM((BPW,), jnp.int32),
            pltpu.VMEM((BPW,), jnp.int32),
            pltpu.VMEM((BPW,), jnp.int32),
            pltpu.VMEM((BPW,), jnp.int32),
            pltpu.VMEM((BPW,), jnp.float32),
            pltpu.VMEM((BPW,), jnp.float32),
            pltpu.VMEM((BPW,), jnp.float32),
            pltpu.SemaphoreType.DMA,
            pltpu.SemaphoreType.DMA,
        ],
    )
    return run(position, prev_stop_loss, action, data_flat, artr_flat,
               date_i, time_i)


# speedup vs baseline: 15.6273x; 15.6273x over previous
import functools

import jax
import jax.numpy as jnp
from jax import lax
from jax.experimental import pallas as pl
from jax.experimental.pallas import tpu as pltpu
from jax.experimental.pallas import tpu_sc as plsc

ATR_MULTIPLE = 2.0
L = 16


def _body(T, C, BPW, DT,
          pos_hbm, prev_hbm, act_hbm, data_hbm, artr_hbm, date_hbm, time_hbm,
          out_hbm,
          pos_v, prev_v, act_v, date_v, time_v, idx1_v, idx2_v, atrv_v,
          refp_v, out_v, sem1, sem2):
    nc = 2
    wid = lax.axis_index("s") * nc + lax.axis_index("c")
    base = wid * BPW

    pltpu.sync_copy(date_hbm.at[pl.ds(base, BPW)], date_v)
    pltpu.sync_copy(time_hbm.at[pl.ds(base, BPW)], time_v)
    pltpu.sync_copy(pos_hbm.at[pl.ds(base, BPW)], pos_v)
    pltpu.sync_copy(act_hbm.at[pl.ds(base, BPW)], act_v)
    pltpu.sync_copy(prev_hbm.at[pl.ds(base, BPW)], prev_v)

    def idx_body(i, carry):
        o = i * L
        d = date_v[pl.ds(o, L)]
        t = time_v[pl.ds(o, L)]
        p = pos_v[pl.ds(o, L)]
        a = act_v[pl.ds(o, L)]
        n = d * T + t
        idx1_v[pl.ds(o, L)] = n
        direction = jnp.sign(p + a)
        chan0 = jnp.where(p == 0.0, 2,
                          jnp.where(direction > 0.0, 0, 1)).astype(jnp.int32)
        idx2_v[pl.ds(o, L)] = chan0 * DT + n
        return carry

    lax.fori_loop(0, BPW // L, idx_body, 0)

    cp1 = pltpu.async_copy(artr_hbm.at[idx1_v], atrv_v, sem1)
    cp2 = pltpu.async_copy(data_hbm.at[idx2_v], refp_v, sem2)
    cp1.wait()
    cp2.wait()

    def out_body(i, carry):
        o = i * L
        p = pos_v[pl.ds(o, L)]
        a = act_v[pl.ds(o, L)]
        prev0 = prev_v[pl.ds(o, L)]
        av = atrv_v[pl.ds(o, L)] * ATR_MULTIPLE + 1.0
        rp = refp_v[pl.ds(o, L)]
        direction = jnp.sign(p + a)
        prev = jnp.where(jnp.isnan(prev0) & (direction != 0.0),
                         -jnp.inf * direction, prev0)
        sp = jnp.where(direction > 0.0,
                       jnp.maximum(prev, rp / av),
                       jnp.minimum(prev, rp * av))
        sp = jnp.where(jnp.isnan(sp) | (direction == 0.0), prev, sp)
        out_v[pl.ds(o, L)] = sp
        return carry

    lax.fori_loop(0, BPW // L, out_body, 0)

    pltpu.sync_copy(out_v, out_hbm.at[pl.ds(base, BPW)])


@jax.jit
def kernel(position, prev_stop_loss, action, data, artr, date_idx, time_idx):
    D, T, C = data.shape
    B = position.shape[0]
    NW = 32
    BPW = B // NW

    data_flat = data[:, :, 1:].transpose(2, 0, 1).reshape((C - 1) * D * T)
    artr_flat = artr.reshape(D * T)
    date_i = date_idx.astype(jnp.int32)
    time_i = time_idx.astype(jnp.int32)

    mesh = plsc.VectorSubcoreMesh(core_axis_name="c", subcore_axis_name="s")
    run = pl.kernel(
        functools.partial(_body, T, C, BPW, D * T),
        out_type=jax.ShapeDtypeStruct((B,), jnp.float32),
        mesh=mesh,
        scratch_types=[
            pltpu.VMEM((BPW,), jnp.float32),
            pltpu.VMEM((BPW,), jnp.float32),
            pltpu.VMEM((BPW,), jnp.float32),
            pltpu.VMEM((BPW,), jnp.int32),
            pltpu.VMEM((BPW,), jnp.int32),
            pltpu.VMEM((BPW,), jnp.int32),
            pltpu.VMEM((BPW,), jnp.int32),
            pltpu.VMEM((BPW,), jnp.float32),
            pltpu.VMEM((BPW,), jnp.float32),
            pltpu.VMEM((BPW,), jnp.float32),
            pltpu.SemaphoreType.DMA,
            pltpu.SemaphoreType.DMA,
        ],
    )
    return run(position, prev_stop_loss, action, data_flat, artr_flat,
               date_i, time_i)


# speedup vs baseline: 45.9823x; 2.9424x over previous
import functools

import jax
import jax.numpy as jnp
from jax import lax
from jax.experimental import pallas as pl
from jax.experimental.pallas import tpu as pltpu
from jax.experimental.pallas import tpu_sc as plsc

ATR_MULTIPLE = 2.0
L = 16


def _body(T, C, BPW,
          pos_hbm, prev_hbm, act_hbm, data_hbm, artr_hbm, date_hbm, time_hbm,
          out_hbm,
          pos_v, prev_v, act_v, date_v, time_v, idx1_v, idx2_v, atrv_v,
          refp_v, out_v, sem1, sem2):
    nc = 2
    wid = lax.axis_index("s") * nc + lax.axis_index("c")
    base = wid * BPW

    pltpu.sync_copy(date_hbm.at[pl.ds(base, BPW)], date_v)
    pltpu.sync_copy(time_hbm.at[pl.ds(base, BPW)], time_v)
    pltpu.sync_copy(pos_hbm.at[pl.ds(base, BPW)], pos_v)
    pltpu.sync_copy(act_hbm.at[pl.ds(base, BPW)], act_v)
    pltpu.sync_copy(prev_hbm.at[pl.ds(base, BPW)], prev_v)

    def idx_body(i, carry):
        o = i * L
        d = date_v[pl.ds(o, L)]
        t = time_v[pl.ds(o, L)]
        p = pos_v[pl.ds(o, L)]
        a = act_v[pl.ds(o, L)]
        idx1_v[pl.ds(o, L)] = d * T + t
        direction = jnp.sign(p + a)
        chan = jnp.where(p == 0.0, 3,
                         jnp.where(direction > 0.0, 1, 2)).astype(jnp.int32)
        idx2_v[pl.ds(o, L)] = (d * C + chan) * T + t
        return carry

    lax.fori_loop(0, BPW // L, idx_body, 0)

    cp1 = pltpu.async_copy(artr_hbm.at[idx1_v], atrv_v, sem1)
    cp2 = pltpu.async_copy(data_hbm.at[idx2_v], refp_v, sem2)
    cp1.wait()
    cp2.wait()

    def out_body(i, carry):
        o = i * L
        p = pos_v[pl.ds(o, L)]
        a = act_v[pl.ds(o, L)]
        prev0 = prev_v[pl.ds(o, L)]
        av = atrv_v[pl.ds(o, L)] * ATR_MULTIPLE + 1.0
        rp = refp_v[pl.ds(o, L)]
        direction = jnp.sign(p + a)
        prev = jnp.where(jnp.isnan(prev0) & (direction != 0.0),
                         -jnp.inf * direction, prev0)
        sp = jnp.where(direction > 0.0,
                       jnp.maximum(prev, rp / av),
                       jnp.minimum(prev, rp * av))
        sp = jnp.where(jnp.isnan(sp) | (direction == 0.0), prev, sp)
        out_v[pl.ds(o, L)] = sp
        return carry

    lax.fori_loop(0, BPW // L, out_body, 0)

    pltpu.sync_copy(out_v, out_hbm.at[pl.ds(base, BPW)])


@jax.jit
def kernel(position, prev_stop_loss, action, data, artr, date_idx, time_idx):
    D, T, C = data.shape
    B = position.shape[0]
    NW = 32
    BPW = B // NW

    data_flat = data.transpose(0, 2, 1).reshape(D * C * T)
    artr_flat = artr.reshape(D * T)
    date_i = date_idx.astype(jnp.int32)
    time_i = time_idx.astype(jnp.int32)

    mesh = plsc.VectorSubcoreMesh(core_axis_name="c", subcore_axis_name="s")
    run = pl.kernel(
        functools.partial(_body, T, C, BPW),
        out_type=jax.ShapeDtypeStruct((B,), jnp.float32),
        mesh=mesh,
        scratch_types=[
            pltpu.VMEM((BPW,), jnp.float32),
            pltpu.VMEM((BPW,), jnp.float32),
            pltpu.VMEM((BPW,), jnp.float32),
            pltpu.VMEM((BPW,), jnp.int32),
            pltpu.VMEM((BPW,), jnp.int32),
            pltpu.VMEM((BPW,), jnp.int32),
            pltpu.VMEM((BPW,), jnp.int32),
            pltpu.VMEM((BPW,), jnp.float32),
            pltpu.VMEM((BPW,), jnp.float32),
            pltpu.VMEM((BPW,), jnp.float32),
            pltpu.SemaphoreType.DMA,
            pltpu.SemaphoreType.DMA,
        ],
    )
    return run(position, prev_stop_loss, action, data_flat, artr_flat,
               date_i, time_i)
